# baseline (speedup 1.0000x reference)
import jax
import jax.numpy as jnp
from jax.experimental import pallas as pl
from jax.experimental.pallas import tpu as pltpu

_T = 4
_NS = 25
_NQ = 75
_N = _NS + _NQ
_R = _T * _N
_D = 512
_M = 512
_THRESH = 0.5
_QK = 0.5
_MARGIN = 0.1


def _l2rows(x):
    ss = jnp.sum(x * x, axis=-1, keepdims=True)
    return x * jax.lax.rsqrt(jnp.maximum(ss, 1e-24))


def _dot_nt(a, b):
    return jax.lax.dot_general(
        a, b, (((1,), (1,)), ((), ())), preferred_element_type=jnp.float32)


def _dot_tn(a, b):
    return jax.lax.dot_general(
        a, b, (((0,), (0,)), ((), ())), preferred_element_type=jnp.float32)


def _dot_nn(a, b):
    return jax.lax.dot_general(
        a, b, (((1,), (0,)), ((), ())), preferred_element_type=jnp.float32)


def _first_argmax_onehot(sim, iota):
    mx = jnp.max(sim, axis=1, keepdims=True)
    idx = jnp.min(jnp.where(sim == mx, iota, _M), axis=1, keepdims=True)
    return (iota == idx).astype(jnp.float32)


def _first_argmin_onehot(sim, iota):
    mn = jnp.min(sim, axis=1, keepdims=True)
    idx = jnp.min(jnp.where(sim == mn, iota, _M), axis=1, keepdims=True)
    return (iota == idx).astype(jnp.float32)


def _fused(es_ref, eq_ref, gs_ref, gq_ref, mk_ref, mv_ref,
           out_ref, lk_ref, lv_ref, emb_ref, glo_ref):
    emb_ref[0:_T * _NS, :] = es_ref[...]
    emb_ref[_T * _NS:, :] = eq_ref[...]
    glo_ref[0:_T * _NS, :] = gs_ref[...]
    glo_ref[_T * _NS:, :] = gq_ref[...]

    ne = _l2rows(emb_ref[...])
    ng = _l2rows(glo_ref[...])
    mk = mk_ref[...]
    mv = mv_ref[...]

    mk_n = _l2rows(mk)
    mv_n = _l2rows(mv)

    iota = jax.lax.broadcasted_iota(jnp.int32, (_R, _M), 1)

    sim_kv = _dot_nt(ne, mk_n)
    score = jnp.where(sim_kv >= _THRESH, sim_kv, 0.0)
    mvu = _l2rows(_QK * mv + ((1.0 - _QK) / _R) * _dot_tn(score, ng))

    sim_vk = _dot_nt(ng, mv_n)
    oh_vk = _first_argmax_onehot(sim_vk, iota)
    mku = _l2rows(_QK * mk + ((1.0 - _QK) / _R) * _dot_tn(oh_vk, ne))

    sim_kv2 = _dot_nt(ne, mku)
    sim_vk2 = _dot_nt(ng, mvu)

    eg = _l2rows(ng + _dot_nn(sim_kv2, mvu))

    for t in range(_T):
        out_ref[t * _N:t * _N + _NS, 0:_D] = ne[t * _NS:(t + 1) * _NS, :]
        out_ref[t * _N + _NS:(t + 1) * _N, 0:_D] = (
            ne[_T * _NS + t * _NQ:_T * _NS + (t + 1) * _NQ, :])
        out_ref[t * _N:t * _N + _NS, _D:2 * _D] = eg[t * _NS:(t + 1) * _NS, :]
        out_ref[t * _N + _NS:(t + 1) * _N, _D:2 * _D] = (
            eg[_T * _NS + t * _NQ:_T * _NS + (t + 1) * _NQ, :])

    ng_sq = jnp.sum(ng * ng, axis=1, keepdims=True)
    ne_sq = jnp.sum(ne * ne, axis=1, keepdims=True)
    mvu_sq = jnp.sum(mvu * mvu, axis=1).reshape(1, _M)
    mku_sq = jnp.sum(mku * mku, axis=1).reshape(1, _M)

    oh_v = _first_argmax_onehot(sim_kv2, iota)
    sel_sq = jnp.sum(oh_v * mvu_sq, axis=1, keepdims=True)
    sel_dot = jnp.sum(oh_v * sim_vk2, axis=1, keepdims=True)
    loss_v_col = sel_sq - 2.0 * sel_dot + ng_sq
    lv_ref[...] = jnp.sum(loss_v_col, axis=0, keepdims=True) / _R

    oh_kmax = _first_argmax_onehot(sim_vk2, iota)
    oh_kmin = _first_argmin_onehot(sim_vk2, iota)
    lmax_col = (jnp.sum(oh_kmax * mku_sq, axis=1, keepdims=True)
                - 2.0 * jnp.sum(oh_kmax * sim_kv2, axis=1, keepdims=True)
                + ne_sq)
    lmin_col = (jnp.sum(oh_kmin * mku_sq, axis=1, keepdims=True)
                - 2.0 * jnp.sum(oh_kmin * sim_kv2, axis=1, keepdims=True)
                + ne_sq)
    diff = jnp.sum(lmax_col - lmin_col, axis=0, keepdims=True) / _R
    lk_ref[...] = jnp.maximum(diff + _MARGIN, 0.0)


def kernel(embedding_support, embedding_query,
           embedding_global_support, embedding_global_query,
           memory_keys, memory_values):
    es2 = embedding_support.reshape(_T * _NS, _D)
    eq2 = embedding_query.reshape(_T * _NQ, _D)
    gs2 = embedding_global_support.reshape(_T * _NS, _D)
    gq2 = embedding_global_query.reshape(_T * _NQ, _D)

    out, lk, lv = pl.pallas_call(
        _fused,
        out_shape=[
            jax.ShapeDtypeStruct((_R, 2 * _D), jnp.float32),
            jax.ShapeDtypeStruct((1, 1), jnp.float32),
            jax.ShapeDtypeStruct((1, 1), jnp.float32),
        ],
        scratch_shapes=[
            pltpu.VMEM((_R, _D), jnp.float32),
            pltpu.VMEM((_R, _D), jnp.float32),
        ],
    )(es2, eq2, gs2, gq2, memory_keys, memory_values)

    return out.reshape(_T, _N, 2 * _D), lk.reshape(()), lv.reshape(())
